# baseline (speedup 1.0000x reference)
import functools

import jax
import jax.numpy as jnp
from jax import lax
from jax.experimental import pallas as pl
from jax.experimental.pallas import tpu as pltpu
from jax.experimental.pallas import tpu_sc as plsc

NODE_NUM = 1000000
EMB_DIM = 32
BATCH = 4096
HIST = 50

LINE = 128
RPL = LINE // EMB_DIM

NC = 2
NS = 16
NW = NC * NS
BPW = BATCH // NW

NBUF = 4

NB = 896
NCHUNK = NODE_NUM // NB
KPW = 36
TAIL0 = NCHUNK * NB
TAILN = NODE_NUM - TAIL0


def _transpose_body(tt_hbm, tail_hbm, out_hbm, vin0, vin1, vout0, vout1,
                    rsem, wsem):
  wid = lax.axis_index("s") * NC + lax.axis_index("c")
  vins = (vin0, vin1)
  vouts = (vout0, vout1)
  rows_lo = lax.iota(jnp.int32, 16)
  rows_hi = rows_lo + 16

  def chunk_of(k):
    return jnp.minimum(wid + NW * k, NCHUNK - 1)

  pltpu.async_copy(tt_hbm.at[:, pl.ds(chunk_of(0) * NB, NB)],
                   vin0.at[:, pl.ds(0, NB)], rsem.at[0])

  def phase(k2, par):
    k = 2 * k2 + par
    c = chunk_of(k)
    pltpu.make_async_copy(tt_hbm.at[:, pl.ds(c * NB, NB)],
                          vins[par].at[:, pl.ds(0, NB)], rsem.at[par]).wait()

    @pl.when(k < KPW - 1)
    def _():
      cn = chunk_of(k + 1)
      pltpu.async_copy(tt_hbm.at[:, pl.ds(cn * NB, NB)],
                       vins[1 - par].at[:, pl.ds(0, NB)], rsem.at[1 - par])

    @pl.when(k >= 2)
    def _():
      pltpu.make_async_copy(vouts[par], out_hbm.at[pl.ds(0, NB * EMB_DIM)],
                            wsem.at[par]).wait()

    @plsc.parallel_loop(0, NB, unroll=4)
    def _(m):
      col = jnp.full((16,), m, jnp.int32)
      v0 = plsc.load_gather(vins[par], [rows_lo, col])
      v1 = plsc.load_gather(vins[par], [rows_hi, col])
      vouts[par][pl.ds(m * EMB_DIM, 16)] = v0
      vouts[par][pl.ds(m * EMB_DIM + 16, 16)] = v1

    pltpu.async_copy(vouts[par], out_hbm.at[pl.ds(c * NB * EMB_DIM,
                                                  NB * EMB_DIM)],
                     wsem.at[par])

  def kbody(k2, carry):
    phase(k2, 0)
    phase(k2, 1)
    return carry

  lax.fori_loop(0, KPW // 2, kbody, 0)

  for par in range(2):
    pltpu.make_async_copy(vouts[par], out_hbm.at[pl.ds(0, NB * EMB_DIM)],
                          wsem.at[par]).wait()

  @pl.when(wid == 0)
  def _():
    pltpu.sync_copy(tail_hbm, vout0.at[pl.ds(0, TAILN * EMB_DIM)])
    pltpu.sync_copy(vout0.at[pl.ds(0, TAILN * EMB_DIM)],
                    out_hbm.at[pl.ds(TAIL0 * EMB_DIM, TAILN * EMB_DIM)])


def _lookup_body(lines_hbm, data_hbm, out_hbm, idx_v, lines_v, offs_v, gbuf_v,
                 acc_v, out_v, sems):
  wid = lax.axis_index("s") * NC + lax.axis_index("c")
  base = wid * BPW

  pltpu.sync_copy(data_hbm.at[pl.ds(base * HIST, BPW * HIST)], idx_v)

  zeros = jnp.zeros((16,), jnp.float32)

  def zbody(b, c):
    acc_v[b, pl.ds(0, 16)] = zeros
    acc_v[b, pl.ds(16, 16)] = zeros
    return c
  lax.fori_loop(0, BPW, zbody, 0, unroll=8)

  lanes50 = lax.iota(jnp.int32, 16) * HIST

  def tbody(j, c):
    for b0 in range(0, BPW, 16):
      vals = plsc.load_gather(idx_v, [lanes50 + (b0 * HIST + j)])
      lines_v[j, pl.ds(b0, 16)] = lax.shift_right_logical(vals, 2)
      offs_v[j, pl.ds(b0, 16)] = (vals & 3) * EMB_DIM
    return c
  lax.fori_loop(0, HIST, tbody, 0)

  def fire(j, slot):
    pltpu.async_copy(lines_hbm.at[lines_v.at[j]], gbuf_v.at[slot],
                     sems.at[slot])

  for j in range(NBUF):
    fire(j, j)

  def loop_j(j, c):
    slot = lax.rem(j, NBUF)
    pltpu.make_async_copy(lines_hbm.at[lines_v.at[j]], gbuf_v.at[slot],
                          sems.at[slot]).wait()
    for b0 in range(0, BPW, 16):
      offv = offs_v[j, pl.ds(b0, 16)]
      for i in range(16):
        b = b0 + i
        off = offv[i]
        plsc.addupdate(acc_v.at[b, pl.ds(0, 16)],
                       gbuf_v[slot, b, pl.ds(off, 16)])
        plsc.addupdate(acc_v.at[b, pl.ds(16, 16)],
                       gbuf_v[slot, b, pl.ds(off + 16, 16)])

    @pl.when(j < HIST - NBUF)
    def _():
      fire(j + NBUF, slot)
    return c

  lax.fori_loop(0, HIST, loop_j, 0)

  scale = jnp.float32(1.0 / HIST)

  def finish(b, c):
    out_v[pl.ds(b * EMB_DIM, 16)] = acc_v[b, pl.ds(0, 16)] * scale
    out_v[pl.ds(b * EMB_DIM + 16, 16)] = acc_v[b, pl.ds(16, 16)] * scale
    return c
  lax.fori_loop(0, BPW, finish, 0, unroll=4)

  pltpu.sync_copy(out_v, out_hbm.at[pl.ds(base * EMB_DIM, BPW * EMB_DIM)])


@jax.jit
def _graph_encode(data, table):
  table_t = table.T
  tail_flat = table[TAIL0:].reshape(TAILN * EMB_DIM)
  data_flat = data.reshape(BATCH * HIST)

  mesh = plsc.VectorSubcoreMesh(
      core_axis_name="c", subcore_axis_name="s", num_cores=NC, num_subcores=NS)

  repack = pl.kernel(
      _transpose_body,
      out_type=jax.ShapeDtypeStruct((NODE_NUM * EMB_DIM,), jnp.float32),
      mesh=mesh,
      scratch_types=[
          pltpu.VMEM((EMB_DIM, NB + 1), jnp.float32),
          pltpu.VMEM((EMB_DIM, NB + 1), jnp.float32),
          pltpu.VMEM((NB * EMB_DIM,), jnp.float32),
          pltpu.VMEM((NB * EMB_DIM,), jnp.float32),
          pltpu.SemaphoreType.DMA((2,)),
          pltpu.SemaphoreType.DMA((2,)),
      ],
      compiler_params=pltpu.CompilerParams(needs_layout_passes=False),
  )
  packed = repack(table_t, tail_flat)
  lines = packed.reshape(NODE_NUM // RPL, LINE)

  lookup = pl.kernel(
      _lookup_body,
      out_type=jax.ShapeDtypeStruct((BATCH * EMB_DIM,), jnp.float32),
      mesh=mesh,
      scratch_types=[
          pltpu.VMEM((BPW * HIST,), jnp.int32),
          pltpu.VMEM((HIST, BPW), jnp.int32),
          pltpu.VMEM((HIST, BPW), jnp.int32),
          pltpu.VMEM((NBUF, BPW, LINE), jnp.float32),
          pltpu.VMEM((BPW, EMB_DIM), jnp.float32),
          pltpu.VMEM((BPW * EMB_DIM,), jnp.float32),
          pltpu.SemaphoreType.DMA((NBUF,)),
      ],
      compiler_params=pltpu.CompilerParams(needs_layout_passes=False),
  )
  return lookup(lines, data_flat).reshape(BATCH, EMB_DIM)


def kernel(data, table):
  return _graph_encode(data, table)


# speedup vs baseline: 1.1520x; 1.1520x over previous
import functools

import jax
import jax.numpy as jnp
from jax import lax
from jax.experimental import pallas as pl
from jax.experimental.pallas import tpu as pltpu
from jax.experimental.pallas import tpu_sc as plsc

NODE_NUM = 1000000
EMB_DIM = 32
BATCH = 4096
HIST = 50

NC = 2
NS = 16
NW = NC * NS
BPW = BATCH // NW

INFLIGHT = 16


def _sc_body(table_hbm, idxt_hbm, out_hbm, idx_v, acc_v, out_v, sem):
  wid = lax.axis_index("s") * NC + lax.axis_index("c")
  base = wid * BPW

  pltpu.sync_copy(idxt_hbm.at[:, pl.ds(base, BPW)], idx_v)

  zeros = jnp.zeros((16,), jnp.float32)

  def zbody(b, c):
    acc_v[b, pl.ds(0, 16)] = zeros
    acc_v[b, pl.ds(16, 16)] = zeros
    return c
  lax.fori_loop(0, BPW, zbody, 0, unroll=8)

  def gather_add(j):
    pltpu.async_copy(table_hbm.at[idx_v.at[j]], acc_v, sem, add=True)

  def drain_one():
    pltpu.make_async_copy(table_hbm.at[idx_v.at[0]], acc_v, sem).wait()

  for j in range(INFLIGHT):
    gather_add(j)
  for j in range(INFLIGHT, HIST):
    drain_one()
    gather_add(j)
  for _ in range(INFLIGHT):
    drain_one()

  scale = jnp.float32(1.0 / HIST)

  def finish(b, c):
    out_v[b, pl.ds(0, 16)] = acc_v[b, pl.ds(0, 16)] * scale
    out_v[b, pl.ds(16, 16)] = acc_v[b, pl.ds(16, 16)] * scale
    return c
  lax.fori_loop(0, BPW, finish, 0, unroll=8)

  pltpu.sync_copy(out_v, out_hbm.at[pl.ds(base, BPW)])


@jax.jit
def _graph_encode(data, table):
  idxt = data.T

  mesh = plsc.VectorSubcoreMesh(
      core_axis_name="c", subcore_axis_name="s", num_cores=NC, num_subcores=NS)
  k = pl.kernel(
      _sc_body,
      out_type=jax.ShapeDtypeStruct((BATCH, EMB_DIM), jnp.float32),
      mesh=mesh,
      scratch_types=[
          pltpu.VMEM((HIST, BPW), jnp.int32),
          pltpu.VMEM((BPW, EMB_DIM), jnp.float32),
          pltpu.VMEM((BPW, EMB_DIM), jnp.float32),
          pltpu.SemaphoreType.DMA,
      ],
      compiler_params=pltpu.CompilerParams(use_tc_tiling_on_sc=False),
  )
  return k(table, idxt)


def kernel(data, table):
  return _graph_encode(data, table)
